# baseline (speedup 1.0000x reference)
import functools

import jax
import jax.numpy as jnp
from jax import lax
from jax.experimental import pallas as pl
from jax.experimental.pallas import tpu as pltpu
from jax.experimental.pallas import tpu_sc as plsc

B = 8
N = 2048
NC = 2
NS = 16
NW = NC * NS
CHUNK = (B * N) // NW
RV = 4
TILE = RV * 16
NTILES = CHUNK // TILE


def _bf16r(x):
    t = x * 65537.0
    return t - (t - x)


def _sc_body(px_hbm, py_hbm, pz_hbm, gx_hbm, gy_hbm, gz_hbm,
             fwd_hbm, bwd_hbm,
             pxr, pyr, pzr, gxr, gyr, gzr, cp, cg, ofv, obv):
    cid = lax.axis_index("c")
    sid = lax.axis_index("s")
    w = sid * NC + cid
    b = w // 4
    base = (w % 4) * CHUNK

    boff = pl.ds(b * N, N)
    pltpu.sync_copy(px_hbm.at[boff], pxr)
    pltpu.sync_copy(py_hbm.at[boff], pyr)
    pltpu.sync_copy(pz_hbm.at[boff], pzr)
    pltpu.sync_copy(gx_hbm.at[boff], gxr)
    pltpu.sync_copy(gy_hbm.at[boff], gyr)
    pltpu.sync_copy(gz_hbm.at[boff], gzr)

    def precompute(xs, ys, zs, dst):
        def step(i, _):
            sl = pl.ds(i * 16, 16)
            x = xs[sl]
            y = ys[sl]
            z = zs[sl]
            dst[0, sl] = -2.0 * _bf16r(x)
            dst[1, sl] = -2.0 * _bf16r(y)
            dst[2, sl] = -2.0 * _bf16r(z)
            dst[3, sl] = x * x + y * y + z * z
            dst[4, sl] = -2.0 * x
            dst[5, sl] = -2.0 * y
            dst[6, sl] = -2.0 * z
            return 0
        lax.fori_loop(0, N // 16, step, 0)

    precompute(pxr, pyr, pzr, cp)
    precompute(gxr, gyr, gzr, cg)

    def direction(rx, ry, rz, coef, out_stage):

        def tile_step(t, _):
            roff = base + t * TILE
            px = [rx[pl.ds(roff + j * 16, 16)] for j in range(RV)]
            py = [ry[pl.ds(roff + j * 16, 16)] for j in range(RV)]
            pz = [rz[pl.ds(roff + j * 16, 16)] for j in range(RV)]
            pbx = [_bf16r(px[j]) for j in range(RV)]
            pby = [_bf16r(py[j]) for j in range(RV)]
            pbz = [_bf16r(pz[j]) for j in range(RV)]
            p2 = [px[j] * px[j] + py[j] * py[j] + pz[j] * pz[j]
                  for j in range(RV)]

            def nstep(i, carry):
                sels, exs = carry
                nb = pl.ds(i * 16, 16)
                cx = coef[0, nb]
                cy = coef[1, nb]
                cz = coef[2, nb]
                cs = coef[3, nb]
                ex = coef[4, nb]
                ey = coef[5, nb]
                ez = coef[6, nb]
                for l in range(16):
                    sx = cx[l]
                    sy = cy[l]
                    sz = cz[l]
                    s2 = cs[l]
                    vx = ex[l]
                    vy = ey[l]
                    vz = ez[l]
                    new_s = []
                    new_e = []
                    for j in range(RV):
                        t_ = s2 + pbx[j] * sx + pby[j] * sy + pbz[j] * sz
                        e_ = s2 + px[j] * vx + py[j] * vy + pz[j] * vz
                        cmp = t_ < sels[j]
                        new_s.append(jnp.minimum(sels[j], t_))
                        new_e.append(jnp.where(cmp, e_, exs[j]))
                    sels, exs = new_s, new_e
                return sels, exs

            init = ([jnp.full((16,), 3.0e38, jnp.float32) for _ in range(RV)],
                    [jnp.full((16,), 3.0e38, jnp.float32) for _ in range(RV)])
            _, exs = lax.fori_loop(0, N // 16, nstep, init)
            for j in range(RV):
                out_stage[pl.ds(t * TILE + j * 16, 16)] = exs[j] + p2[j]
            return 0

        lax.fori_loop(0, NTILES, tile_step, 0)

    direction(pxr, pyr, pzr, cg, ofv)
    direction(gxr, gyr, gzr, cp, obv)
    pltpu.sync_copy(ofv, fwd_hbm.at[b, pl.ds(base, CHUNK)])
    pltpu.sync_copy(obv, bwd_hbm.at[b, pl.ds(base, CHUNK)])


@functools.partial(jax.jit, static_argnames=())
def _min_d2(pred3, gt3):
    mesh = plsc.VectorSubcoreMesh(core_axis_name="c", subcore_axis_name="s")
    f = pl.kernel(
        _sc_body,
        mesh=mesh,
        out_type=[
            jax.ShapeDtypeStruct((B, N), jnp.float32),
            jax.ShapeDtypeStruct((B, N), jnp.float32),
        ],
        scratch_types=[
            pltpu.VMEM((N,), jnp.float32),
            pltpu.VMEM((N,), jnp.float32),
            pltpu.VMEM((N,), jnp.float32),
            pltpu.VMEM((N,), jnp.float32),
            pltpu.VMEM((N,), jnp.float32),
            pltpu.VMEM((N,), jnp.float32),
            pltpu.VMEM((8, N), jnp.float32),
            pltpu.VMEM((8, N), jnp.float32),
            pltpu.VMEM((CHUNK,), jnp.float32),
            pltpu.VMEM((CHUNK,), jnp.float32),
        ],
    )
    return f(pred3[:, 0, :].reshape(-1), pred3[:, 1, :].reshape(-1),
             pred3[:, 2, :].reshape(-1), gt3[:, 0, :].reshape(-1),
             gt3[:, 1, :].reshape(-1), gt3[:, 2, :].reshape(-1))


def _finish_body(thresh_ref, f_ref, b_ref, o_ref):
    th = thresh_ref[0]

    def accsum(ref):
        acc = jnp.zeros((B, 128), jnp.float32)
        for i in range(N // 128):
            d2 = ref[:, i * 128:(i + 1) * 128]
            el = jnp.sqrt(jnp.maximum(d2, 0.0) + 1e-8)
            acc = acc + jnp.maximum(el - th, 0.0)
        return jnp.sum(acc)

    o_ref[0, 0] = (accsum(f_ref) + accsum(b_ref)) / (B * N)


def _finish(fwd_d2, bwd_d2, thresh):
    out = pl.pallas_call(
        _finish_body,
        out_shape=jax.ShapeDtypeStruct((1, 1), jnp.float32),
        in_specs=[
            pl.BlockSpec(memory_space=pltpu.SMEM),
            pl.BlockSpec(memory_space=pltpu.VMEM),
            pl.BlockSpec(memory_space=pltpu.VMEM),
        ],
        out_specs=pl.BlockSpec(memory_space=pltpu.SMEM),
    )(thresh.reshape(1), fwd_d2, bwd_d2)
    return out[0, 0]


def kernel(predict_pc_6, gt_pc_6, thresh):
    pred3 = predict_pc_6[:, :3, :]
    gt3 = gt_pc_6[:, :3, :]
    fwd_d2, bwd_d2 = _min_d2(pred3, gt3)
    return _finish(fwd_d2, bwd_d2, thresh.astype(jnp.float32))


# speedup vs baseline: 2.6866x; 2.6866x over previous
import jax
import jax.numpy as jnp
from jax import lax
from jax.experimental import pallas as pl
from jax.experimental.pallas import tpu as pltpu
from jax.experimental.pallas import tpu_sc as plsc

B = 8
N = 2048
NC = 2
NS = 16
NW = NC * NS
K_SC = 2
NB_TC = B - K_SC
WPB = NW // K_SC
ROWS = N // WPB
RV = 4
TILE = RV * 16
NTILES = ROWS // TILE
MB = 512


def _bf16r(x):
    t = x * 65537.0
    return t - (t - x)


def _sc_body(px_hbm, py_hbm, pz_hbm, gx_hbm, gy_hbm, gz_hbm,
             fwd_hbm, bwd_hbm,
             pxr, pyr, pzr, gxr, gyr, gzr, cp, cg, ofv, obv):
    cid = lax.axis_index("c")
    sid = lax.axis_index("s")
    w = sid * NC + cid
    b = w // WPB
    base = (w % WPB) * ROWS

    boff = pl.ds(b * N, N)
    pltpu.sync_copy(px_hbm.at[boff], pxr)
    pltpu.sync_copy(py_hbm.at[boff], pyr)
    pltpu.sync_copy(pz_hbm.at[boff], pzr)
    pltpu.sync_copy(gx_hbm.at[boff], gxr)
    pltpu.sync_copy(gy_hbm.at[boff], gyr)
    pltpu.sync_copy(gz_hbm.at[boff], gzr)

    def precompute(xs, ys, zs, dst):
        def step(i, _):
            sl = pl.ds(i * 16, 16)
            x = xs[sl]
            y = ys[sl]
            z = zs[sl]
            dst[0, sl] = -2.0 * _bf16r(x)
            dst[1, sl] = -2.0 * _bf16r(y)
            dst[2, sl] = -2.0 * _bf16r(z)
            dst[3, sl] = x * x + y * y + z * z
            dst[4, sl] = -2.0 * x
            dst[5, sl] = -2.0 * y
            dst[6, sl] = -2.0 * z
            return 0
        lax.fori_loop(0, N // 16, step, 0)

    precompute(pxr, pyr, pzr, cp)
    precompute(gxr, gyr, gzr, cg)

    def direction(rx, ry, rz, coef, out_stage):
        def tile_step(t, _):
            roff = base + t * TILE
            px = [rx[pl.ds(roff + j * 16, 16)] for j in range(RV)]
            py = [ry[pl.ds(roff + j * 16, 16)] for j in range(RV)]
            pz = [rz[pl.ds(roff + j * 16, 16)] for j in range(RV)]
            pbx = [_bf16r(px[j]) for j in range(RV)]
            pby = [_bf16r(py[j]) for j in range(RV)]
            pbz = [_bf16r(pz[j]) for j in range(RV)]
            p2 = [px[j] * px[j] + py[j] * py[j] + pz[j] * pz[j]
                  for j in range(RV)]

            def nstep(i, carry):
                sels, exs = carry
                nb = pl.ds(i * 16, 16)
                cx = coef[0, nb]
                cy = coef[1, nb]
                cz = coef[2, nb]
                cs = coef[3, nb]
                ex = coef[4, nb]
                ey = coef[5, nb]
                ez = coef[6, nb]
                for l in range(16):
                    sx = cx[l]
                    sy = cy[l]
                    sz = cz[l]
                    s2 = cs[l]
                    vx = ex[l]
                    vy = ey[l]
                    vz = ez[l]
                    new_s = []
                    new_e = []
                    for j in range(RV):
                        t_ = s2 + pbx[j] * sx + pby[j] * sy + pbz[j] * sz
                        e_ = s2 + px[j] * vx + py[j] * vy + pz[j] * vz
                        cmp = t_ < sels[j]
                        new_s.append(jnp.minimum(sels[j], t_))
                        new_e.append(jnp.where(cmp, e_, exs[j]))
                    sels, exs = new_s, new_e
                return sels, exs

            init = ([jnp.full((16,), 3.0e38, jnp.float32) for _ in range(RV)],
                    [jnp.full((16,), 3.0e38, jnp.float32) for _ in range(RV)])
            _, exs = lax.fori_loop(0, N // 16, nstep, init)
            for j in range(RV):
                out_stage[pl.ds(t * TILE + j * 16, 16)] = exs[j] + p2[j]
            return 0

        lax.fori_loop(0, NTILES, tile_step, 0)

    direction(pxr, pyr, pzr, cg, ofv)
    direction(gxr, gyr, gzr, cp, obv)
    pltpu.sync_copy(ofv, fwd_hbm.at[b, pl.ds(base, ROWS)])
    pltpu.sync_copy(obv, bwd_hbm.at[b, pl.ds(base, ROWS)])


def _min_d2_sc(pred3, gt3):
    mesh = plsc.VectorSubcoreMesh(core_axis_name="c", subcore_axis_name="s")
    f = pl.kernel(
        _sc_body,
        mesh=mesh,
        out_type=[
            jax.ShapeDtypeStruct((K_SC, N), jnp.float32),
            jax.ShapeDtypeStruct((K_SC, N), jnp.float32),
        ],
        scratch_types=[
            pltpu.VMEM((N,), jnp.float32),
            pltpu.VMEM((N,), jnp.float32),
            pltpu.VMEM((N,), jnp.float32),
            pltpu.VMEM((N,), jnp.float32),
            pltpu.VMEM((N,), jnp.float32),
            pltpu.VMEM((N,), jnp.float32),
            pltpu.VMEM((8, N), jnp.float32),
            pltpu.VMEM((8, N), jnp.float32),
            pltpu.VMEM((ROWS,), jnp.float32),
            pltpu.VMEM((ROWS,), jnp.float32),
        ],
    )
    return f(pred3[:, 0, :].reshape(-1), pred3[:, 1, :].reshape(-1),
             pred3[:, 2, :].reshape(-1), gt3[:, 0, :].reshape(-1),
             gt3[:, 1, :].reshape(-1), gt3[:, 2, :].reshape(-1))


def _tc_dir_body(rows_ref, opp_ref, o_ref):
    pr = rows_ref[0]
    px = pr[:, 0:1]
    py = pr[:, 1:2]
    pz = pr[:, 2:3]
    g = opp_ref[0]
    gx = g[0:1, :]
    gy = g[1:2, :]
    gz = g[2:3, :]
    cbx = -2.0 * _bf16r(gx)
    cby = -2.0 * _bf16r(gy)
    cbz = -2.0 * _bf16r(gz)
    g2 = gx * gx + gy * gy + gz * gz
    pbx = _bf16r(px)
    pby = _bf16r(py)
    pbz = _bf16r(pz)
    t = g2 + pbx * cbx + pby * cby + pbz * cbz
    e = g2 + px * (-2.0 * gx) + py * (-2.0 * gy) + pz * (-2.0 * gz)
    rowmin = jnp.min(t, axis=1, keepdims=True)
    esel = jnp.min(jnp.where(t == rowmin, e, 3.0e38), axis=1, keepdims=True)
    p2 = px * px + py * py + pz * pz
    o_ref[...] = (esel + p2).reshape(1, MB, 1)


def _min_d2_tc(rows_T, opp):
    nm = N // MB
    return pl.pallas_call(
        _tc_dir_body,
        grid=(NB_TC, nm),
        in_specs=[
            pl.BlockSpec((1, MB, 3), lambda b, m: (b, m, 0)),
            pl.BlockSpec((1, 3, N), lambda b, m: (b, 0, 0)),
        ],
        out_specs=pl.BlockSpec((1, MB, 1), lambda b, m: (b * nm + m, 0, 0)),
        out_shape=jax.ShapeDtypeStruct((NB_TC * nm, MB, 1), jnp.float32),
    )(rows_T, opp).reshape(NB_TC, N)


def _finish_body(thresh_ref, sf_ref, sb_ref, tf_ref, tb_ref, o_ref):
    th = thresh_ref[0]

    def acc_sc(ref):
        acc = jnp.zeros((K_SC, 128), jnp.float32)
        for i in range(N // 128):
            d2 = ref[:, i * 128:(i + 1) * 128]
            el = jnp.sqrt(jnp.maximum(d2, 0.0) + 1e-8)
            acc = acc + jnp.maximum(el - th, 0.0)
        return jnp.sum(acc)

    def acc_tc(ref):
        acc = jnp.zeros((NB_TC, 128), jnp.float32)
        for i in range(N // 128):
            d2 = ref[:, i * 128:(i + 1) * 128]
            el = jnp.sqrt(jnp.maximum(d2, 0.0) + 1e-8)
            acc = acc + jnp.maximum(el - th, 0.0)
        return jnp.sum(acc)

    fwd = acc_sc(sf_ref) + acc_tc(tf_ref)
    bwd = acc_sc(sb_ref) + acc_tc(tb_ref)
    o_ref[0, 0] = (fwd + bwd) / (B * N)


def _finish(sc_f, sc_b, tc_f, tc_b, thresh):
    out = pl.pallas_call(
        _finish_body,
        out_shape=jax.ShapeDtypeStruct((1, 1), jnp.float32),
        in_specs=[
            pl.BlockSpec(memory_space=pltpu.SMEM),
            pl.BlockSpec(memory_space=pltpu.VMEM),
            pl.BlockSpec(memory_space=pltpu.VMEM),
            pl.BlockSpec(memory_space=pltpu.VMEM),
            pl.BlockSpec(memory_space=pltpu.VMEM),
        ],
        out_specs=pl.BlockSpec(memory_space=pltpu.SMEM),
    )(thresh.reshape(1), sc_f, sc_b, tc_f, tc_b)
    return out[0, 0]


def kernel(predict_pc_6, gt_pc_6, thresh):
    pred3 = predict_pc_6[:, :3, :]
    gt3 = gt_pc_6[:, :3, :]
    sc_f, sc_b = _min_d2_sc(pred3, gt3)
    pred3_tc = pred3[K_SC:]
    gt3_tc = gt3[K_SC:]
    tc_f = _min_d2_tc(jnp.transpose(pred3_tc, (0, 2, 1)), gt3_tc)
    tc_b = _min_d2_tc(jnp.transpose(gt3_tc, (0, 2, 1)), pred3_tc)
    return _finish(sc_f, sc_b, tc_f, tc_b, thresh.astype(jnp.float32))
